# baseline (device time: 20100 ns/iter reference)
import jax
import jax.numpy as jnp
from jax import lax
from jax.experimental import pallas as pl
from jax.experimental.pallas import tpu as pltpu


def kernel(x):
    m_per, n = x.shape
    n_half = n // 2

    def body(
        x_ref,
        out_ref,
        xv_peer,
        xv_mine,
        comm_ref,
        local_bf,
        in_sems,
        out_sem,
        send_sem,
        recv_sem,
    ):
        my_x = lax.axis_index("x")
        my_y = lax.axis_index("y")
        my_z = lax.axis_index("z")
        peer_z = 1 - my_z

        barrier_sem = pltpu.get_barrier_semaphore()
        pl.semaphore_signal(
            barrier_sem,
            inc=1,
            device_id=(my_x, my_y, peer_z),
            device_id_type=pl.DeviceIdType.MESH,
        )

        dma_peer = pltpu.make_async_copy(
            x_ref.at[:, pl.ds(peer_z * n_half, n_half)], xv_peer, in_sems.at[0]
        )
        dma_peer.start()
        dma_mine = pltpu.make_async_copy(
            x_ref.at[:, pl.ds(my_z * n_half, n_half)], xv_mine, in_sems.at[1]
        )
        dma_mine.start()

        dma_peer.wait()
        comm_ref[:, :] = xv_peer[:, :].astype(jnp.bfloat16)

        pl.semaphore_wait(barrier_sem, 1)

        rdma = pltpu.make_async_remote_copy(
            src_ref=comm_ref,
            dst_ref=out_ref.at[pl.ds(my_z * m_per, m_per), :],
            send_sem=send_sem,
            recv_sem=recv_sem,
            device_id=(my_x, my_y, peer_z),
            device_id_type=pl.DeviceIdType.MESH,
        )
        rdma.start()

        dma_mine.wait()
        local_bf[:, :] = xv_mine[:, :].astype(jnp.bfloat16)
        dma_out = pltpu.make_async_copy(
            local_bf, out_ref.at[pl.ds(my_z * m_per, m_per), :], out_sem
        )
        dma_out.start()
        dma_out.wait()

        rdma.wait()

    x = pltpu.with_memory_space_constraint(x, pltpu.MemorySpace.HBM)
    return pl.pallas_call(
        body,
        out_shape=pltpu.MemorySpace.HBM((2 * m_per, n_half), jnp.bfloat16),
        in_specs=[pl.BlockSpec(memory_space=pltpu.MemorySpace.HBM)],
        out_specs=pl.BlockSpec(memory_space=pltpu.MemorySpace.HBM),
        scratch_shapes=[
            pltpu.VMEM((m_per, n_half), jnp.float32),
            pltpu.VMEM((m_per, n_half), jnp.float32),
            pltpu.VMEM((m_per, n_half), jnp.bfloat16),
            pltpu.VMEM((m_per, n_half), jnp.bfloat16),
            pltpu.SemaphoreType.DMA((2,)),
            pltpu.SemaphoreType.DMA,
            pltpu.SemaphoreType.DMA,
            pltpu.SemaphoreType.DMA,
        ],
        compiler_params=pltpu.CompilerParams(collective_id=0),
    )(x)


# device time: 18261 ns/iter; 1.1007x vs baseline; 1.1007x over previous
import jax
import jax.numpy as jnp
from jax import lax
from jax.experimental import pallas as pl
from jax.experimental.pallas import tpu as pltpu


def kernel(x):
    m_per, n = x.shape
    n_half = n // 2

    def body(
        x_ref,
        out_ref,
        xv_peer,
        xv_mine,
        comm_ref,
        local_bf,
        in_sems,
        out_sem,
        send_sem,
        recv_sem,
    ):
        my_x = lax.axis_index("x")
        my_y = lax.axis_index("y")
        my_z = lax.axis_index("z")
        peer_z = 1 - my_z

        barrier_sem = pltpu.get_barrier_semaphore()
        pl.semaphore_signal(
            barrier_sem,
            inc=1,
            device_id=(my_x, my_y, peer_z),
            device_id_type=pl.DeviceIdType.MESH,
        )

        dma_peer = pltpu.make_async_copy(
            x_ref.at[:, pl.ds(peer_z * n_half, n_half)], xv_peer, in_sems.at[0]
        )
        dma_peer.start()
        dma_mine = pltpu.make_async_copy(
            x_ref.at[:, pl.ds(my_z * n_half, n_half)], xv_mine, in_sems.at[1]
        )
        dma_mine.start()

        dma_peer.wait()
        comm_ref[:, :] = xv_peer[:, :].astype(jnp.bfloat16)

        pl.semaphore_wait(barrier_sem, 1)

        rdma = pltpu.make_async_remote_copy(
            src_ref=comm_ref,
            dst_ref=out_ref.at[pl.ds(my_z * m_per, m_per), :],
            send_sem=send_sem,
            recv_sem=recv_sem,
            device_id=(my_x, my_y, peer_z),
            device_id_type=pl.DeviceIdType.MESH,
        )
        rdma.start()

        dma_mine.wait()
        local_bf[:, :] = xv_mine[:, :].astype(jnp.bfloat16)
        dma_out = pltpu.make_async_copy(
            local_bf, out_ref.at[pl.ds(my_z * m_per, m_per), :], out_sem
        )
        dma_out.start()
        dma_out.wait()

        rdma.wait()

    x = pltpu.with_memory_space_constraint(x, pltpu.MemorySpace.HBM)
    return pl.pallas_call(
        body,
        out_shape=jax.ShapeDtypeStruct((2 * m_per, n_half), jnp.bfloat16),
        in_specs=[pl.BlockSpec(memory_space=pltpu.MemorySpace.HBM)],
        out_specs=pl.BlockSpec(memory_space=pltpu.MemorySpace.HBM),
        scratch_shapes=[
            pltpu.VMEM((m_per, n_half), jnp.float32),
            pltpu.VMEM((m_per, n_half), jnp.float32),
            pltpu.VMEM((m_per, n_half), jnp.bfloat16),
            pltpu.VMEM((m_per, n_half), jnp.bfloat16),
            pltpu.SemaphoreType.DMA((2,)),
            pltpu.SemaphoreType.DMA,
            pltpu.SemaphoreType.DMA,
            pltpu.SemaphoreType.DMA,
        ],
        compiler_params=pltpu.CompilerParams(collective_id=0),
    )(x)


# device time: 17986 ns/iter; 1.1175x vs baseline; 1.0153x over previous
import jax
import jax.numpy as jnp
from jax import lax
from jax.experimental import pallas as pl
from jax.experimental.pallas import tpu as pltpu


def kernel(x):
    m_per, n = x.shape
    n_half = n // 2
    m_chunk = m_per // 2

    def body(
        x_ref,
        out_ref,
        xv_peer,
        xv_mine,
        comm_ref,
        local_bf,
        in_sems,
        out_sem,
        send_sems,
        recv_sems,
    ):
        my_x = lax.axis_index("x")
        my_y = lax.axis_index("y")
        my_z = lax.axis_index("z")
        peer_z = 1 - my_z

        barrier_sem = pltpu.get_barrier_semaphore()
        pl.semaphore_signal(
            barrier_sem,
            inc=1,
            device_id=(my_x, my_y, peer_z),
            device_id_type=pl.DeviceIdType.MESH,
        )

        dma_a = pltpu.make_async_copy(
            x_ref.at[pl.ds(0, m_chunk), pl.ds(peer_z * n_half, n_half)],
            xv_peer.at[pl.ds(0, m_chunk), :],
            in_sems.at[0],
        )
        dma_a.start()
        dma_b = pltpu.make_async_copy(
            x_ref.at[pl.ds(m_chunk, m_chunk), pl.ds(peer_z * n_half, n_half)],
            xv_peer.at[pl.ds(m_chunk, m_chunk), :],
            in_sems.at[1],
        )
        dma_b.start()

        dma_a.wait()
        comm_ref[pl.ds(0, m_chunk), :] = xv_peer[pl.ds(0, m_chunk), :].astype(
            jnp.bfloat16
        )

        pl.semaphore_wait(barrier_sem, 1)

        rdma_a = pltpu.make_async_remote_copy(
            src_ref=comm_ref.at[pl.ds(0, m_chunk), :],
            dst_ref=out_ref.at[pl.ds(my_z * m_per, m_chunk), :],
            send_sem=send_sems.at[0],
            recv_sem=recv_sems.at[0],
            device_id=(my_x, my_y, peer_z),
            device_id_type=pl.DeviceIdType.MESH,
        )
        rdma_a.start()

        dma_b.wait()
        comm_ref[pl.ds(m_chunk, m_chunk), :] = xv_peer[
            pl.ds(m_chunk, m_chunk), :
        ].astype(jnp.bfloat16)
        rdma_b = pltpu.make_async_remote_copy(
            src_ref=comm_ref.at[pl.ds(m_chunk, m_chunk), :],
            dst_ref=out_ref.at[pl.ds(my_z * m_per + m_chunk, m_chunk), :],
            send_sem=send_sems.at[1],
            recv_sem=recv_sems.at[1],
            device_id=(my_x, my_y, peer_z),
            device_id_type=pl.DeviceIdType.MESH,
        )
        rdma_b.start()

        dma_mine = pltpu.make_async_copy(
            x_ref.at[:, pl.ds(my_z * n_half, n_half)], xv_mine, in_sems.at[2]
        )
        dma_mine.start()
        dma_mine.wait()
        local_bf[:, :] = xv_mine[:, :].astype(jnp.bfloat16)
        dma_out = pltpu.make_async_copy(
            local_bf, out_ref.at[pl.ds(my_z * m_per, m_per), :], out_sem
        )
        dma_out.start()
        dma_out.wait()

        rdma_a.wait()
        rdma_b.wait()

    x = pltpu.with_memory_space_constraint(x, pltpu.MemorySpace.HBM)
    return pl.pallas_call(
        body,
        out_shape=jax.ShapeDtypeStruct((2 * m_per, n_half), jnp.bfloat16),
        in_specs=[pl.BlockSpec(memory_space=pltpu.MemorySpace.HBM)],
        out_specs=pl.BlockSpec(memory_space=pltpu.MemorySpace.HBM),
        scratch_shapes=[
            pltpu.VMEM((m_per, n_half), jnp.float32),
            pltpu.VMEM((m_per, n_half), jnp.float32),
            pltpu.VMEM((m_per, n_half), jnp.bfloat16),
            pltpu.VMEM((m_per, n_half), jnp.bfloat16),
            pltpu.SemaphoreType.DMA((3,)),
            pltpu.SemaphoreType.DMA,
            pltpu.SemaphoreType.DMA((2,)),
            pltpu.SemaphoreType.DMA((2,)),
        ],
        compiler_params=pltpu.CompilerParams(collective_id=0),
    )(x)
